# baseline (speedup 1.0000x reference)
import functools
import jax
import jax.numpy as jnp
from jax import lax
from jax.experimental import pallas as pl
from jax.experimental.pallas import tpu as pltpu
from jax.experimental.pallas import tpu_sc as plsc

D = 128
M = 100000
B = 1024
S = 50
TOPK = 5
NW = 16
TILE_N = 6272
TPAD = NW * TILE_N
NVEC = TILE_N // 16
ROWS_PER_BLOCK = 20000
M_BLOCKS = M // ROWS_PER_BLOCK
G_BLOCK = 256
B_BLOCKS = B // G_BLOCK

_HI = jax.lax.Precision.HIGHEST
_NEG = jnp.float32(-jnp.inf)


def _red_min(v):
    s = v[0]
    for k in range(1, 16):
        s = jnp.minimum(s, v[k])
    return s


def _red_max(v):
    s = v[0]
    for k in range(1, 16):
        s = jnp.maximum(s, v[k])
    return s

def _sc_scores(scores_hbm, fb_hbm, mem_hbm,
               nscores_hbm, idx_hbm, best_hbm,
               buf, nbuf, fbv, stagef, stagei, lvals, lidxs, rowbuf, acc,
               sh_vals, sh_idxs):
    wid = lax.axis_index("s")
    base = wid * TILE_N
    ii = lax.broadcasted_iota(jnp.int32, (16,), 0)

    pltpu.sync_copy(scores_hbm.at[pl.ds(base, TILE_N)], buf)
    pltpu.sync_copy(fb_hbm, fbv)

    def _minbody(j8, carry):
        m, mi = carry
        for u in range(8):
            j = j8 * 8 + u
            v = buf[pl.ds(j * 16, 16)]
            g = base + j * 16 + ii
            upd = v < m
            m = jnp.where(upd, v, m)
            mi = jnp.where(upd, g, mi)
        return m, mi

    m0 = jnp.full((16,), jnp.inf, jnp.float32)
    i0 = jnp.zeros((16,), jnp.int32)
    m, mi = lax.fori_loop(0, NVEC // 8, _minbody, (m0, i0))
    mval = _red_min(m)
    midx = _red_min(jnp.where(m == mval, mi, M))

    tile_tv, tile_ti = [], []
    for _p in range(TOPK):
        def _maxbody(j8, carry, _chosen=tuple(tile_ti)):
            m, mi = carry
            for u in range(8):
                j = j8 * 8 + u
                v = buf[pl.ds(j * 16, 16)]
                g = base + j * 16 + ii
                v = jnp.where(g < M, v, _NEG)
                for t_q in _chosen:
                    v = jnp.where(g == t_q, _NEG, v)
                upd = v >= m
                m = jnp.where(upd, v, m)
                mi = jnp.where(upd, g, mi)
            return m, mi

        mx, gi = lax.fori_loop(0, NVEC // 8, _maxbody,
                               (jnp.full((16,), _NEG), i0))
        mxv = _red_max(mx)
        tile_tv.append(mxv)
        tile_ti.append(_red_max(jnp.where(mx == mxv, gi, -1)))

    sv = jnp.where(ii == 0, mval, _NEG)
    si = jnp.where(ii == 0, midx, -1)
    for k in range(TOPK):
        sv = jnp.where(ii == 1 + k, tile_tv[k], sv)
        si = jnp.where(ii == 1 + k, tile_ti[k], si)
    stagef[...] = sv
    stagei[...] = si
    pltpu.sync_copy(stagef, sh_vals.at[pl.ds(wid * 16, 16)])
    pltpu.sync_copy(stagei, sh_idxs.at[pl.ds(wid * 16, 16)])
    plsc.subcore_barrier()
    pltpu.sync_copy(sh_vals, lvals)
    pltpu.sync_copy(sh_idxs, lidxs)

    def _amin(r, carry):
        bv, bi = carry
        val = lvals[pl.ds(r * 16, 16)][0]
        idx = lidxs[pl.ds(r * 16, 16)][0]
        upd = (val < bv) | ((val == bv) & (idx < bi))
        return jnp.where(upd, val, bv), jnp.where(upd, idx, bi)

    _, min_idx = lax.fori_loop(0, NW, _amin, (jnp.float32(jnp.inf),
                                              jnp.int32(M)))

    fbs = fbv[...][0]

    def _nsbody(j8, _):
        for u in range(8):
            j = j8 * 8 + u
            v = buf[pl.ds(j * 16, 16)]
            g = base + j * 16 + ii
            nbuf[pl.ds(j * 16, 16)] = jnp.where(g == min_idx, fbs, v)
        return 0

    lax.fori_loop(0, NVEC // 8, _nsbody, 0)
    pltpu.sync_copy(nbuf, nscores_hbm.at[pl.ds(base, TILE_N)])

    glob_t = []
    for _p in range(TOPK):
        def _gmax(r, carry, _chosen=tuple(glob_t)):
            m, mi = carry
            v = lvals[pl.ds(r * 16, 16)]
            g = lidxs[pl.ds(r * 16, 16)]
            v = jnp.where((ii >= 1) & (ii <= TOPK), v, _NEG)
            for t_q in _chosen:
                v = jnp.where(g == t_q, _NEG, v)
            upd = (v > m) | ((v == m) & (g > mi))
            return jnp.where(upd, v, m), jnp.where(upd, g, mi)

        mx, gi = lax.fori_loop(0, NW, _gmax,
                               (jnp.full((16,), _NEG), jnp.full((16,), -1)))
        mxv = _red_max(mx)
        glob_t.append(_red_max(jnp.where(mx == mxv, gi, -1)))

    @pl.when(wid == 0)
    def _tile0_outputs():
        for c in range(D // 16):
            acc[pl.ds(c * 16, 16)] = jnp.zeros((16,), jnp.float32)
        for k in range(TOPK):
            tk = jnp.minimum(jnp.maximum(glob_t[k], 0), M - 1)
            pltpu.sync_copy(mem_hbm.at[tk], rowbuf)
            for c in range(D // 16):
                acc[pl.ds(c * 16, 16)] = (acc[pl.ds(c * 16, 16)]
                                          + rowbuf[pl.ds(c * 16, 16)]
                                          * (1.0 / TOPK))
        pltpu.sync_copy(acc, best_hbm)
        oi = jnp.where(ii == 0, min_idx, 0)
        for k in range(TOPK):
            oi = jnp.where(ii == 1 + k, glob_t[k], oi)
        stagei[...] = oi
        pltpu.sync_copy(stagei, idx_hbm)


def _encode(pooled, w1, b1, w2, b2, g, o):
    h = jax.nn.silu(jnp.dot(pooled, w1, precision=_HI) + b1)
    h = jnp.dot(h, w2, precision=_HI) + b2
    mu = jnp.mean(h, axis=-1, keepdims=True)
    var = jnp.mean((h - mu) ** 2, axis=-1, keepdims=True)
    h = (h - mu) * jax.lax.rsqrt(var + 1e-5)
    return h * g + o


def _copy_kernel(mem_ref, out_ref):
    out_ref[...] = mem_ref[...]


def _dense_kernel(ci_ref, best_ref, idx_ref, w1_ref, b1_ref, w2_ref, b2_ref,
                  g_ref, o_ref, sw1_ref, sb1_ref, sw2_ref, sb2_ref, pw1_ref,
                  pb1_ref, pw2_ref, pb2_ref, newmem_in_ref,
                  strat_ref, ei_ref, newmem_out_ref, enc0_ref, sem):
    i = pl.program_id(0)
    pooled = jnp.mean(ci_ref[:], axis=0)
    encoded = _encode(pooled, w1_ref[:], b1_ref[:], w2_ref[:], b2_ref[:],
                      g_ref[:], o_ref[:])

    @pl.when(i == 0)
    def _save_enc0():
        enc0_ref[...] = encoded[0:1, :]

    best = jnp.broadcast_to(best_ref[:], encoded.shape)
    combined = jnp.concatenate([best, encoded], axis=-1)
    h = jax.nn.silu(jnp.dot(combined, sw1_ref[:], precision=_HI) + sb1_ref[:])
    strategy = jnp.tanh(jnp.dot(h, sw2_ref[:], precision=_HI) + sb2_ref[:])
    strat_ref[...] = strategy
    h2 = jax.nn.silu(jnp.dot(strategy, pw1_ref[:], precision=_HI) + pb1_ref[:])
    ei = jax.nn.sigmoid(jnp.dot(h2, pw2_ref[:], precision=_HI) + pb2_ref[:])
    ei_ref[...] = ei

    @pl.when(i == B_BLOCKS - 1)
    def _scatter_row():
        cp = pltpu.make_async_copy(
            enc0_ref, newmem_out_ref.at[pl.ds(idx_ref[0], 1), :], sem)
        cp.start()
        cp.wait()


def kernel(current_input, performance_feedback, experience_memory,
           memory_scores, enc_w1, enc_b1, enc_w2, enc_b2, ln_scale, ln_offset,
           sg_w1, sg_b1, sg_w2, sg_b2, pp_w1, pp_b1, pp_w2, pp_b2):
    b1 = jnp.reshape(enc_b1, (1, D))
    b2 = jnp.reshape(enc_b2, (1, D))
    g = jnp.reshape(ln_scale, (1, D))
    o = jnp.reshape(ln_offset, (1, D))
    sb1 = jnp.reshape(sg_b1, (1, 2 * D))
    sb2 = jnp.reshape(sg_b2, (1, D))
    pb1 = jnp.reshape(pp_b1, (1, D))
    pb2 = jnp.reshape(pp_b2, (1, 1))
    scores_pad = jnp.pad(memory_scores, (0, TPAD - M),
                         constant_values=jnp.inf)
    fb16 = jnp.broadcast_to(jnp.reshape(performance_feedback, (1,)), (16,))

    sc_fn = pl.kernel(
        _sc_scores,
        out_type=[
            jax.ShapeDtypeStruct((TPAD,), jnp.float32),
            jax.ShapeDtypeStruct((16,), jnp.int32),
            jax.ShapeDtypeStruct((D,), jnp.float32),
        ],
        mesh=plsc.VectorSubcoreMesh(core_axis_name="c", subcore_axis_name="s", num_cores=1),
        scratch_types=[
            pltpu.VMEM((TILE_N,), jnp.float32),
            pltpu.VMEM((TILE_N,), jnp.float32),
            pltpu.VMEM((16,), jnp.float32),
            pltpu.VMEM((16,), jnp.float32),
            pltpu.VMEM((16,), jnp.int32),
            pltpu.VMEM((NW * 16,), jnp.float32),
            pltpu.VMEM((NW * 16,), jnp.int32),
            pltpu.VMEM((D,), jnp.float32),
            pltpu.VMEM((D,), jnp.float32),
            pltpu.VMEM_SHARED((NW * 16,), jnp.float32),
            pltpu.VMEM_SHARED((NW * 16,), jnp.int32),
        ],
    )
    nscores_pad, idx16, best_experiences = sc_fn(
        scores_pad, fb16, experience_memory)
    new_scores = nscores_pad[:M]
    best_sum = jnp.reshape(best_experiences, (1, D))

    new_memory0 = pl.pallas_call(
        _copy_kernel,
        grid=(M_BLOCKS,),
        in_specs=[pl.BlockSpec((ROWS_PER_BLOCK, D), lambda i: (i, 0))],
        out_specs=pl.BlockSpec((ROWS_PER_BLOCK, D), lambda i: (i, 0)),
        out_shape=jax.ShapeDtypeStruct((M, D), jnp.float32),
    )(experience_memory)

    ci_t = jnp.transpose(current_input, (1, 0, 2))
    strategy, expected_improvement, new_memory = pl.pallas_call(
        _dense_kernel,
        grid=(B_BLOCKS,),
        in_specs=[
            pl.BlockSpec((S, G_BLOCK, D), lambda i: (0, i, 0)),
            pl.BlockSpec((1, D), lambda i: (0, 0)),
            pl.BlockSpec(memory_space=pltpu.SMEM),
        ] + [pl.BlockSpec(memory_space=pltpu.VMEM)] * 14
          + [pl.BlockSpec(memory_space=pl.ANY)],
        out_specs=[
            pl.BlockSpec((G_BLOCK, D), lambda i: (i, 0)),
            pl.BlockSpec((G_BLOCK, 1), lambda i: (i, 0)),
            pl.BlockSpec(memory_space=pl.ANY),
        ],
        out_shape=[
            jax.ShapeDtypeStruct((B, D), jnp.float32),
            jax.ShapeDtypeStruct((B, 1), jnp.float32),
            jax.ShapeDtypeStruct((M, D), jnp.float32),
        ],
        scratch_shapes=[
            pltpu.VMEM((1, D), jnp.float32),
            pltpu.SemaphoreType.DMA,
        ],
        input_output_aliases={17: 2},
    )(ci_t, best_sum, idx16, enc_w1, b1, enc_w2, b2, g, o,
      sg_w1, sb1, sg_w2, sb2, pp_w1, pb1, pp_w2, pb2, new_memory0)

    return (strategy, expected_improvement, best_experiences, new_memory,
            new_scores)


# speedup vs baseline: 1.2185x; 1.2185x over previous
import jax
import jax.numpy as jnp
from jax import lax
from jax.experimental import pallas as pl
from jax.experimental.pallas import tpu as pltpu

D = 128
M = 100000
B = 1024
S = 50
TOPK = 5
SROWS = 782
MPAD = SROWS * 128
ROWS_PER_BLOCK = 20000
M_BLOCKS = M // ROWS_PER_BLOCK
G_BLOCK = 256
B_BLOCKS = B // G_BLOCK

_HI = jax.lax.Precision.HIGHEST


def _encode(pooled, w1, b1, w2, b2, g, o):
    h = jax.nn.silu(jnp.dot(pooled, w1, precision=_HI) + b1)
    h = jnp.dot(h, w2, precision=_HI) + b2
    mu = jnp.mean(h, axis=-1, keepdims=True)
    var = jnp.mean((h - mu) ** 2, axis=-1, keepdims=True)
    h = (h - mu) * jax.lax.rsqrt(var + 1e-5)
    return h * g + o


def _memory_kernel(scores_ref, fb_ref, x0_ref, w1_ref, b1_ref, w2_ref,
                   b2_ref, g_ref, o_ref, mem_ref,
                   out_ref, new_scores_ref, best_ref,
                   idx_ref, enc0_ref):
    i = pl.program_id(0)

    @pl.when(i == 0)
    def _scores_work():
        scores = scores_ref[:]
        iota = (lax.broadcasted_iota(jnp.int32, scores.shape, 0) * 128
                + lax.broadcasted_iota(jnp.int32, scores.shape, 1))
        mn = jnp.min(scores)
        min_idx = jnp.min(jnp.where(scores == mn, iota, M))
        idx_ref[0] = min_idx
        work = jnp.where(iota < M, scores, -jnp.inf)
        for k in range(TOPK):
            mx = jnp.max(work)
            t = jnp.max(jnp.where(work == mx, iota, -1))
            idx_ref[1 + k] = t
            work = jnp.where(iota == t, -jnp.inf, work)
        new_scores_ref[:] = jnp.where(iota == min_idx, fb_ref[0], scores)
        pooled0 = jnp.mean(x0_ref[:], axis=0, keepdims=True)
        enc0_ref[:] = _encode(pooled0, w1_ref[:], b1_ref[:], w2_ref[:],
                              b2_ref[:], g_ref[:], o_ref[:])
        best_ref[...] = jnp.zeros((1, D), jnp.float32)

    base = i * ROWS_PER_BLOCK
    out_ref[...] = mem_ref[...]
    mi = idx_ref[0] - base

    @pl.when((mi >= 0) & (mi < ROWS_PER_BLOCK))
    def _scatter():
        out_ref[pl.ds(mi, 1), :] = enc0_ref[...]

    for k in range(TOPK):
        t = idx_ref[1 + k] - base

        @pl.when((t >= 0) & (t < ROWS_PER_BLOCK))
        def _gather():
            best_ref[...] += mem_ref[pl.ds(t, 1), :] * (1.0 / TOPK)


def _dense_kernel(ci_ref, best_ref, w1_ref, b1_ref, w2_ref, b2_ref, g_ref,
                  o_ref, sw1_ref, sb1_ref, sw2_ref, sb2_ref, pw1_ref, pb1_ref,
                  pw2_ref, pb2_ref, strat_ref, ei_ref):
    pooled = jnp.mean(ci_ref[:], axis=0)
    encoded = _encode(pooled, w1_ref[:], b1_ref[:], w2_ref[:], b2_ref[:],
                      g_ref[:], o_ref[:])
    best = jnp.broadcast_to(best_ref[:], encoded.shape)
    combined = jnp.concatenate([best, encoded], axis=-1)
    h = jax.nn.silu(jnp.dot(combined, sw1_ref[:], precision=_HI) + sb1_ref[:])
    strategy = jnp.tanh(jnp.dot(h, sw2_ref[:], precision=_HI) + sb2_ref[:])
    strat_ref[...] = strategy
    h2 = jax.nn.silu(jnp.dot(strategy, pw1_ref[:], precision=_HI) + pb1_ref[:])
    ei = jax.nn.sigmoid(jnp.dot(h2, pw2_ref[:], precision=_HI) + pb2_ref[:])
    ei_ref[...] = ei


def kernel(current_input, performance_feedback, experience_memory,
           memory_scores, enc_w1, enc_b1, enc_w2, enc_b2, ln_scale, ln_offset,
           sg_w1, sg_b1, sg_w2, sg_b2, pp_w1, pp_b1, pp_w2, pp_b2):
    fb = jnp.reshape(performance_feedback, (1,))
    b1 = jnp.reshape(enc_b1, (1, D))
    b2 = jnp.reshape(enc_b2, (1, D))
    g = jnp.reshape(ln_scale, (1, D))
    o = jnp.reshape(ln_offset, (1, D))
    sb1 = jnp.reshape(sg_b1, (1, 2 * D))
    sb2 = jnp.reshape(sg_b2, (1, D))
    pb1 = jnp.reshape(pp_b1, (1, D))
    pb2 = jnp.reshape(pp_b2, (1, 1))
    x0 = current_input[0]
    scores2d = jnp.reshape(
        jnp.pad(memory_scores, (0, MPAD - M), constant_values=jnp.inf),
        (SROWS, 128))

    new_memory, new_scores2d, best_sum = pl.pallas_call(
        _memory_kernel,
        grid=(M_BLOCKS,),
        in_specs=[
            pl.BlockSpec((SROWS, 128), lambda i: (0, 0)),
            pl.BlockSpec(memory_space=pltpu.SMEM),
            pl.BlockSpec((S, D), lambda i: (0, 0)),
            pl.BlockSpec((D, D), lambda i: (0, 0)),
            pl.BlockSpec((1, D), lambda i: (0, 0)),
            pl.BlockSpec((D, D), lambda i: (0, 0)),
            pl.BlockSpec((1, D), lambda i: (0, 0)),
            pl.BlockSpec((1, D), lambda i: (0, 0)),
            pl.BlockSpec((1, D), lambda i: (0, 0)),
            pl.BlockSpec((ROWS_PER_BLOCK, D), lambda i: (i, 0)),
        ],
        out_specs=[
            pl.BlockSpec((ROWS_PER_BLOCK, D), lambda i: (i, 0)),
            pl.BlockSpec((SROWS, 128), lambda i: (0, 0)),
            pl.BlockSpec((1, D), lambda i: (0, 0)),
        ],
        out_shape=[
            jax.ShapeDtypeStruct((M, D), jnp.float32),
            jax.ShapeDtypeStruct((SROWS, 128), jnp.float32),
            jax.ShapeDtypeStruct((1, D), jnp.float32),
        ],
        scratch_shapes=[
            pltpu.SMEM((1 + TOPK,), jnp.int32),
            pltpu.VMEM((1, D), jnp.float32),
        ],
    )(scores2d, fb, x0, enc_w1, b1, enc_w2, b2, g, o, experience_memory)

    new_scores = jnp.reshape(new_scores2d, (MPAD,))[:M]

    ci_t = jnp.transpose(current_input, (1, 0, 2))
    strategy, expected_improvement = pl.pallas_call(
        _dense_kernel,
        grid=(B_BLOCKS,),
        in_specs=[
            pl.BlockSpec((S, G_BLOCK, D), lambda i: (0, i, 0)),
            pl.BlockSpec((1, D), lambda i: (0, 0)),
        ] + [pl.BlockSpec(memory_space=pltpu.VMEM)] * 14,
        out_specs=[
            pl.BlockSpec((G_BLOCK, D), lambda i: (i, 0)),
            pl.BlockSpec((G_BLOCK, 1), lambda i: (i, 0)),
        ],
        out_shape=[
            jax.ShapeDtypeStruct((B, D), jnp.float32),
            jax.ShapeDtypeStruct((B, 1), jnp.float32),
        ],
    )(ci_t, best_sum, enc_w1, b1, enc_w2, b2, g, o,
      sg_w1, sb1, sg_w2, sb2, pp_w1, pb1, pp_w2, pb2)

    best_experiences = jnp.reshape(best_sum, (D,))
    return (strategy, expected_improvement, best_experiences, new_memory,
            new_scores)
